# Initial kernel scaffold; baseline (speedup 1.0000x reference)
import jax
import jax.numpy as jnp
from jax.experimental import pallas as pl


def kernel(embeddings, source_id, target_id, target_distance, confidence):
    raise NotImplementedError("write your pallas kernel here")


# speedup vs baseline: 1.1896x; 1.1896x over previous
import functools

import jax
import jax.numpy as jnp
from jax import lax
from jax.experimental import pallas as pl
from jax.experimental.pallas import tpu as pltpu
from jax.experimental.pallas import tpu_sc as plsc

_N_NODES = 10000
_D = 128
_N_EDGES = 320000
_NW = 32
_E_PER_W = _N_EDGES // _NW
_CHUNK = 80
_N_CHUNKS = _E_PER_W // _CHUNK
_G = _CHUNK // 16

_SC_SCRATCH = [
    pltpu.VMEM((_E_PER_W,), jnp.int32),
    pltpu.VMEM((_E_PER_W,), jnp.int32),
    pltpu.VMEM((_E_PER_W,), jnp.float32),
    pltpu.VMEM((_E_PER_W,), jnp.float32),
    pltpu.VMEM((_CHUNK,), jnp.int32),
    pltpu.VMEM((_CHUNK,), jnp.int32),
    pltpu.VMEM((_CHUNK, _D), jnp.float32),
    pltpu.VMEM((_CHUNK, _D), jnp.float32),
    pltpu.VMEM((16,), jnp.float32),
    pltpu.SemaphoreType.DMA,
    pltpu.SemaphoreType.DMA,
]


def _sc_edge_loss_body(emb_h, sid_h, tid_h, td_h, cf_h, out_h,
                  sid_v, tid_v, td_v, cf_v, sidc, tidc, sbuf, tbuf, acc_v,
                  sem_s, sem_t):
    wid = lax.axis_index("s") * 2 + lax.axis_index("c")
    base = wid * _E_PER_W
    pltpu.sync_copy(sid_h.at[pl.ds(base, _E_PER_W)], sid_v)
    pltpu.sync_copy(tid_h.at[pl.ds(base, _E_PER_W)], tid_v)
    pltpu.sync_copy(td_h.at[pl.ds(base, _E_PER_W)], td_v)
    pltpu.sync_copy(cf_h.at[pl.ds(base, _E_PER_W)], cf_v)

    lane = lax.iota(jnp.int32, 16)

    def chunk_body(c, acc):
        off = pl.multiple_of(c * _CHUNK, _CHUNK)
        for k in range(_G):
            sidc[pl.ds(k * 16, 16)] = sid_v[pl.ds(off + k * 16, 16)]
            tidc[pl.ds(k * 16, 16)] = tid_v[pl.ds(off + k * 16, 16)]
        cp_s = pltpu.async_copy(emb_h.at[sidc], sbuf, sem_s)
        cp_t = pltpu.async_copy(emb_h.at[tidc], tbuf, sem_t)
        cp_s.wait()
        cp_t.wait()
        for g in range(_G):
            idx_e = lane + g * 16

            def f_body(f, a):
                idx_f = jnp.zeros((16,), jnp.int32) + f
                sv = plsc.load_gather(sbuf, [idx_e, idx_f])
                tv = plsc.load_gather(tbuf, [idx_e, idx_f])
                d = sv - tv
                return a + d * d

            ss = lax.fori_loop(0, _D, f_body, jnp.zeros((16,), jnp.float32))
            ss = jnp.maximum(ss, 1e-30)
            i = plsc.bitcast(ss, jnp.int32)
            i = jnp.int32(0x5F3759DF) - lax.shift_right_logical(i, 1)
            r = plsc.bitcast(i, jnp.float32)
            for _ in range(3):
                r = r * (1.5 - 0.5 * ss * r * r)
            dist = ss * r
            tdv = td_v[pl.ds(off + g * 16, 16)]
            cfv = cf_v[pl.ds(off + g * 16, 16)]
            e = dist - tdv
            acc = acc + e * e * cfv
        return acc

    acc = lax.fori_loop(0, _N_CHUNKS, chunk_body, jnp.zeros((16,), jnp.float32))
    acc_v[...] = acc
    pltpu.sync_copy(acc_v, out_h.at[wid])


@functools.cache
def _build_sc_edge_loss():
    mesh = plsc.VectorSubcoreMesh(
        core_axis_name="c", subcore_axis_name="s", num_cores=2, num_subcores=16
    )
    return pl.kernel(
        _sc_edge_loss_body,
        out_type=jax.ShapeDtypeStruct((_NW, 16), jnp.float32),
        mesh=mesh,
        scratch_types=_SC_SCRATCH,
        compiler_params=pltpu.CompilerParams(needs_layout_passes=False),
    )


def _tc_mean(x_ref, o_ref):
    o_ref[...] = jnp.sum(x_ref[...]).reshape(1, 1) * (1.0 / _N_EDGES)


def kernel(embeddings, source_id, target_id, target_distance, confidence):
    parts = _build_sc_edge_loss()(
        embeddings,
        source_id.astype(jnp.int32),
        target_id.astype(jnp.int32),
        target_distance,
        confidence,
    )
    out = pl.pallas_call(
        _tc_mean,
        out_shape=jax.ShapeDtypeStruct((1, 1), jnp.float32),
    )(parts)
    return out[0, 0]


# speedup vs baseline: 1.3527x; 1.1371x over previous
import functools

import jax
import jax.numpy as jnp
from jax import lax
from jax.experimental import pallas as pl
from jax.experimental.pallas import tpu as pltpu
from jax.experimental.pallas import tpu_sc as plsc

_N_NODES = 10000
_D = 128
_N_EDGES = 320000
_NW = 32
_E_PER_W = _N_EDGES // _NW
_CHUNK = 80
_N_CHUNKS = _E_PER_W // _CHUNK
_G = _CHUNK // 16
_FU = 4

_SC_SCRATCH = [
    pltpu.VMEM((_E_PER_W,), jnp.int32),
    pltpu.VMEM((_E_PER_W,), jnp.int32),
    pltpu.VMEM((_E_PER_W,), jnp.float32),
    pltpu.VMEM((_E_PER_W,), jnp.float32),
    pltpu.VMEM((_CHUNK, _D), jnp.float32),
    pltpu.VMEM((_CHUNK, _D), jnp.float32),
    pltpu.VMEM((_CHUNK, _D), jnp.float32),
    pltpu.VMEM((_CHUNK, _D), jnp.float32),
    pltpu.VMEM((16,), jnp.float32),
    pltpu.SemaphoreType.DMA,
    pltpu.SemaphoreType.DMA,
]


def _sqrt16(x):
    i = plsc.bitcast(x, jnp.int32)
    i = jnp.int32(0x5F3759DF) - lax.shift_right_logical(i, 1)
    r = plsc.bitcast(i, jnp.float32)
    for _ in range(3):
        r = r * (1.5 - 0.5 * x * r * r)
    return x * r


def _sc_edge_loss_body(emb_h, sid_h, tid_h, td_h, cf_h, out_h,
                       sid_v, tid_v, td_v, cf_v,
                       sbuf0, tbuf0, sbuf1, tbuf1, acc_v,
                       sem0, sem1):
    wid = lax.axis_index("s") * 2 + lax.axis_index("c")
    base = wid * _E_PER_W
    pltpu.sync_copy(sid_h.at[pl.ds(base, _E_PER_W)], sid_v)
    pltpu.sync_copy(tid_h.at[pl.ds(base, _E_PER_W)], tid_v)
    pltpu.sync_copy(td_h.at[pl.ds(base, _E_PER_W)], td_v)
    pltpu.sync_copy(cf_h.at[pl.ds(base, _E_PER_W)], cf_v)

    lane = lax.iota(jnp.int32, 16)
    idx_es = [lane + g * 16 for g in range(_G)]

    def issue(c, sbuf, tbuf, sem):
        off = pl.multiple_of(c * _CHUNK, 8)
        pltpu.async_copy(emb_h.at[sid_v.at[pl.ds(off, _CHUNK)]], sbuf, sem)
        pltpu.async_copy(emb_h.at[tid_v.at[pl.ds(off, _CHUNK)]], tbuf, sem)

    def wait_slot(sbuf, tbuf, sem):
        dummy = emb_h.at[pl.ds(0, _CHUNK)]
        pltpu.make_async_copy(dummy, sbuf, sem).wait()
        pltpu.make_async_copy(dummy, tbuf, sem).wait()

    def compute(c, sbuf, tbuf, acc):
        off = c * _CHUNK

        def f_body(i, accs):
            accs = list(accs)
            for u in range(_FU):
                idx_f = jnp.zeros((16,), jnp.int32) + (i * _FU + u)
                for g in range(_G):
                    sv = plsc.load_gather(sbuf, [idx_es[g], idx_f])
                    tv = plsc.load_gather(tbuf, [idx_es[g], idx_f])
                    d = sv - tv
                    accs[g] = accs[g] + d * d
            return tuple(accs)

        zero = jnp.zeros((16,), jnp.float32)
        accs = lax.fori_loop(0, _D // _FU, f_body, (zero,) * _G)
        for g in range(_G):
            ss = jnp.maximum(accs[g], 1e-30)
            dist = _sqrt16(ss)
            tdv = td_v[pl.ds(off + g * 16, 16)]
            cfv = cf_v[pl.ds(off + g * 16, 16)]
            e = dist - tdv
            acc = acc + e * e * cfv
        return acc

    issue(0, sbuf0, tbuf0, sem0)
    issue(1, sbuf1, tbuf1, sem1)

    def pair_body(p, acc):
        c0 = p * 2
        wait_slot(sbuf0, tbuf0, sem0)
        acc = compute(c0, sbuf0, tbuf0, acc)
        issue(c0 + 2, sbuf0, tbuf0, sem0)
        wait_slot(sbuf1, tbuf1, sem1)
        acc = compute(c0 + 1, sbuf1, tbuf1, acc)

        @pl.when(p < (_N_CHUNKS - 3) // 2)
        def _():
            issue(c0 + 3, sbuf1, tbuf1, sem1)

        return acc

    acc = lax.fori_loop(0, (_N_CHUNKS - 1) // 2, pair_body,
                        jnp.zeros((16,), jnp.float32))
    wait_slot(sbuf0, tbuf0, sem0)
    acc = compute(_N_CHUNKS - 1, sbuf0, tbuf0, acc)

    acc_v[...] = acc
    pltpu.sync_copy(acc_v, out_h.at[wid])


@functools.cache
def _build_sc_edge_loss():
    mesh = plsc.VectorSubcoreMesh(
        core_axis_name="c", subcore_axis_name="s", num_cores=2, num_subcores=16
    )
    return pl.kernel(
        _sc_edge_loss_body,
        out_type=jax.ShapeDtypeStruct((_NW, 16), jnp.float32),
        mesh=mesh,
        scratch_types=_SC_SCRATCH,
        compiler_params=pltpu.CompilerParams(needs_layout_passes=False),
    )


def _tc_mean(x_ref, o_ref):
    o_ref[...] = jnp.sum(x_ref[...]).reshape(1, 1) * (1.0 / _N_EDGES)


def kernel(embeddings, source_id, target_id, target_distance, confidence):
    parts = _build_sc_edge_loss()(
        embeddings,
        source_id.astype(jnp.int32),
        target_id.astype(jnp.int32),
        target_distance,
        confidence,
    )
    out = pl.pallas_call(
        _tc_mean,
        out_shape=jax.ShapeDtypeStruct((1, 1), jnp.float32),
    )(parts)
    return out[0, 0]


# speedup vs baseline: 3.3610x; 2.4847x over previous
import functools

import jax
import jax.numpy as jnp
from jax import lax
from jax.experimental import pallas as pl
from jax.experimental.pallas import tpu as pltpu
from jax.experimental.pallas import tpu_sc as plsc

_N_NODES = 10000
_D = 128
_N_EDGES = 320000
_NW = 32
_E_PER_W = _N_EDGES // _NW
_CHUNK = 80
_N_CHUNKS = _E_PER_W // _CHUNK
_G = _CHUNK // 16

_SC_SCRATCH = [
    pltpu.VMEM((_E_PER_W,), jnp.int32),
    pltpu.VMEM((_E_PER_W,), jnp.int32),
    pltpu.VMEM((_E_PER_W,), jnp.float32),
    pltpu.VMEM((_E_PER_W,), jnp.float32),
    pltpu.VMEM((_CHUNK, _D), jnp.float32),
    pltpu.VMEM((_CHUNK, _D), jnp.float32),
    pltpu.VMEM((_CHUNK, _D), jnp.float32),
    pltpu.VMEM((_CHUNK, _D), jnp.float32),
    pltpu.VMEM((16,), jnp.float32),
    pltpu.SemaphoreType.DMA,
    pltpu.SemaphoreType.DMA,
]


def _sqrt16(x):
    i = plsc.bitcast(x, jnp.int32)
    i = jnp.int32(0x5F3759DF) - lax.shift_right_logical(i, 1)
    r = plsc.bitcast(i, jnp.float32)
    for _ in range(3):
        r = r * (1.5 - 0.5 * x * r * r)
    return x * r


def _sc_edge_loss_body(emb_h, sid_h, tid_h, td_h, cf_h, out_h,
                       sid_v, tid_v, td_v, cf_v,
                       sbuf0, tbuf0, sbuf1, tbuf1, acc_v,
                       sem0, sem1):
    wid = lax.axis_index("s") * 2 + lax.axis_index("c")
    base = wid * _E_PER_W
    pltpu.sync_copy(sid_h.at[pl.ds(base, _E_PER_W)], sid_v)
    pltpu.sync_copy(tid_h.at[pl.ds(base, _E_PER_W)], tid_v)
    pltpu.sync_copy(td_h.at[pl.ds(base, _E_PER_W)], td_v)
    pltpu.sync_copy(cf_h.at[pl.ds(base, _E_PER_W)], cf_v)

    lane = lax.iota(jnp.int32, 16)

    def issue(c, sbuf, tbuf, sem):
        off = pl.multiple_of(c * _CHUNK, 8)
        pltpu.async_copy(emb_h.at[sid_v.at[pl.ds(off, _CHUNK)]], sbuf, sem)
        pltpu.async_copy(emb_h.at[tid_v.at[pl.ds(off, _CHUNK)]], tbuf, sem)

    def wait_slot(sbuf, tbuf, sem):
        dummy = emb_h.at[pl.ds(0, _CHUNK)]
        pltpu.make_async_copy(dummy, sbuf, sem).wait()
        pltpu.make_async_copy(dummy, tbuf, sem).wait()

    def compute(c, sbuf, tbuf, acc):
        off = c * _CHUNK

        def g_body(g, acc):
            e0 = g * 16
            ssvec = jnp.zeros((16,), jnp.float32)
            for j in range(16):
                a = jnp.zeros((16,), jnp.float32)
                for k in range(_D // 16):
                    sv = sbuf[e0 + j, pl.ds(k * 16, 16)]
                    tv = tbuf[e0 + j, pl.ds(k * 16, 16)]
                    d = sv - tv
                    a = a + d * d
                ss = jnp.sum(a)
                ssvec = jnp.where(lane == j, ss, ssvec)
            ssvec = jnp.maximum(ssvec, 1e-30)
            dist = _sqrt16(ssvec)
            tdv = td_v[pl.ds(off + e0, 16)]
            cfv = cf_v[pl.ds(off + e0, 16)]
            e = dist - tdv
            return acc + e * e * cfv

        return lax.fori_loop(0, _G, g_body, acc)

    issue(0, sbuf0, tbuf0, sem0)
    issue(1, sbuf1, tbuf1, sem1)

    def pair_body(p, acc):
        c0 = p * 2
        wait_slot(sbuf0, tbuf0, sem0)
        acc = compute(c0, sbuf0, tbuf0, acc)
        issue(c0 + 2, sbuf0, tbuf0, sem0)
        wait_slot(sbuf1, tbuf1, sem1)
        acc = compute(c0 + 1, sbuf1, tbuf1, acc)

        @pl.when(p < (_N_CHUNKS - 3) // 2)
        def _():
            issue(c0 + 3, sbuf1, tbuf1, sem1)

        return acc

    acc = lax.fori_loop(0, (_N_CHUNKS - 1) // 2, pair_body,
                        jnp.zeros((16,), jnp.float32))
    wait_slot(sbuf0, tbuf0, sem0)
    acc = compute(_N_CHUNKS - 1, sbuf0, tbuf0, acc)

    acc_v[...] = acc
    pltpu.sync_copy(acc_v, out_h.at[wid])


@functools.cache
def _build_sc_edge_loss():
    mesh = plsc.VectorSubcoreMesh(
        core_axis_name="c", subcore_axis_name="s", num_cores=2, num_subcores=16
    )
    return pl.kernel(
        _sc_edge_loss_body,
        out_type=jax.ShapeDtypeStruct((_NW, 16), jnp.float32),
        mesh=mesh,
        scratch_types=_SC_SCRATCH,
        compiler_params=pltpu.CompilerParams(needs_layout_passes=False),
    )


def _tc_mean(x_ref, o_ref):
    o_ref[...] = jnp.sum(x_ref[...]).reshape(1, 1) * (1.0 / _N_EDGES)


def kernel(embeddings, source_id, target_id, target_distance, confidence):
    parts = _build_sc_edge_loss()(
        embeddings,
        source_id.astype(jnp.int32),
        target_id.astype(jnp.int32),
        target_distance,
        confidence,
    )
    out = pl.pallas_call(
        _tc_mean,
        out_shape=jax.ShapeDtypeStruct((1, 1), jnp.float32),
    )(parts)
    return out[0, 0]
